# baseline (speedup 1.0000x reference)
import functools
import jax
import jax.numpy as jnp
from jax import lax
from jax.experimental import pallas as pl
from jax.experimental.pallas import tpu as pltpu
from jax.experimental.pallas import tpu_sc as plsc

N = 10000
NPAD = 10112
E = 320000
NC = 2
NS = 16
NW = NC * NS
EW = E // NW
CHUNK = 100
NCH = EW // CHUNK
EWP = 10080
ECH = 84
G = 12
NG = EWP // ECH // G
RPT = NPAD // NS


@functools.lru_cache(maxsize=None)
def _sc_mesh():
    return plsc.VectorSubcoreMesh(core_axis_name="c", subcore_axis_name="s",
                                  num_cores=NC, num_subcores=NS)


def _memset_zero(ref, nrows, width):
    z = jnp.zeros((16,), jnp.float32)

    def body(i, _):
        for k in range(width // 16):
            ref[i, pl.ds(k * 16, 16)] = z
        return 0

    lax.fori_loop(0, nrows, body, 0)


def _zero_acc_slice(zsrc, acc, s):
    for j in range(8):
        pltpu.sync_copy(zsrc.at[pl.ds(0, 72)],
                        acc.at[pl.ds(s * RPT + j * 72, 72)])
    pltpu.sync_copy(zsrc.at[pl.ds(0, 56)],
                    acc.at[pl.ds(s * RPT + 576, 56)])


def _edge_scatter_body(D, y_hbm, src_hbm, dst_hbm, out_hbm,
                       srcg_a, srcg_b, dstg_a, dstg_b,
                       rows0, rows1, rows2, acc,
                       sem_ga, sem_gb,
                       sem_g0, sem_g1, sem_g2,
                       sem_s0, sem_s1, sem_s2):
    c = lax.axis_index("c")
    s = lax.axis_index("s")
    wid = s * NC + c
    rows = (rows0, rows1, rows2)
    sem_g = (sem_g0, sem_g1, sem_g2)
    sem_s = (sem_s0, sem_s1, sem_s2)

    def fire_idx(g, srcg, dstg, sem):
        pltpu.async_copy(src_hbm.at[wid, g], srcg, sem)
        pltpu.async_copy(dst_hbm.at[wid, g], dstg, sem)

    def wait_idx(g, srcg, dstg, sem):
        pltpu.make_async_copy(src_hbm.at[wid, g], srcg, sem).wait()
        pltpu.make_async_copy(dst_hbm.at[wid, g], dstg, sem).wait()

    fire_idx(0, srcg_a, dstg_a, sem_ga)

    _memset_zero(rows0, 72, D)
    _zero_acc_slice(rows0, acc, s)
    plsc.subcore_barrier()
    wait_idx(0, srcg_a, dstg_a, sem_ga)

    def fire_g(j, srcg):
        b = j % 3
        pltpu.async_copy(y_hbm.at[srcg.at[j % G]], rows[b], sem_g[b])

    def wait_g(j, srcg):
        b = j % 3
        pltpu.make_async_copy(y_hbm.at[srcg.at[j % G]], rows[b],
                              sem_g[b]).wait()

    def fire_s(j, dstg):
        b = j % 3
        pltpu.async_copy(rows[b], acc.at[dstg.at[j % G]], sem_s[b],
                         add=True)

    def wait_s(j, dstg):
        b = j % 3
        pltpu.make_async_copy(rows[b], acc.at[dstg.at[j % G]],
                              sem_s[b]).wait()

    def _noop():
        pass

    def process_group(srcg, dstg, next_srcg, hook2=_noop, hook6=_noop):
        for j in range(G):
            if j == 2:
                hook2()
            if j == 6:
                hook6()
            wait_g(j, srcg)
            fire_s(j, dstg)
            wait_s(j, dstg)
            if j + 3 < G:
                fire_g(j + 3, srcg)
            else:
                fire_g(j + 3, next_srcg)

    fire_g(0, srcg_a)
    fire_g(1, srcg_a)
    fire_g(2, srcg_a)

    def gpair(g2, _):
        g = g2 * 2
        fire_idx(g + 1, srcg_b, dstg_b, sem_gb)
        process_group(srcg_a, dstg_a, srcg_b,
                      hook2=lambda: wait_idx(g + 1, srcg_b, dstg_b, sem_gb))

        def fire_next_a():
            @pl.when(g2 < NG // 2 - 1)
            def _():
                fire_idx(g + 2, srcg_a, dstg_a, sem_ga)

        def wait_next_a():
            @pl.when(g2 < NG // 2 - 1)
            def _():
                wait_idx(g + 2, srcg_a, dstg_a, sem_ga)

        process_group(srcg_b, dstg_b, srcg_a,
                      hook2=fire_next_a, hook6=wait_next_a)
        return 0

    lax.fori_loop(0, NG // 2, gpair, 0)

    wait_g(0, srcg_a)
    wait_g(1, srcg_a)
    wait_g(2, srcg_a)
    plsc.subcore_barrier()

    pltpu.sync_copy(acc.at[pl.ds(s * RPT, RPT)],
                    out_hbm.at[c, pl.ds(s * RPT, RPT)])


@functools.lru_cache(maxsize=None)
def _make_edge_scatter(D):
    body = functools.partial(_edge_scatter_body, D)
    return pl.kernel(
        body,
        out_type=jax.ShapeDtypeStruct((NC, NPAD, D), jnp.float32),
        mesh=_sc_mesh(),
        scratch_types=[
            pltpu.VMEM((G, ECH), jnp.int32),
            pltpu.VMEM((G, ECH), jnp.int32),
            pltpu.VMEM((G, ECH), jnp.int32),
            pltpu.VMEM((G, ECH), jnp.int32),
            pltpu.VMEM((ECH, D), jnp.float32),
            pltpu.VMEM((ECH, D), jnp.float32),
            pltpu.VMEM((ECH, D), jnp.float32),
            pltpu.VMEM_SHARED((NPAD, D), jnp.float32),
            pltpu.SemaphoreType.DMA,
            pltpu.SemaphoreType.DMA,
            pltpu.SemaphoreType.DMA,
            pltpu.SemaphoreType.DMA,
            pltpu.SemaphoreType.DMA,
            pltpu.SemaphoreType.DMA,
            pltpu.SemaphoreType.DMA,
            pltpu.SemaphoreType.DMA,
        ],
    )


def _deg_body(dst_hbm, out_hbm, dst_v, ones_v, acc):
    c = lax.axis_index("c")
    s = lax.axis_index("s")
    wid = s * NC + c

    pltpu.sync_copy(dst_hbm.at[wid], dst_v)

    _memset_zero(ones_v, 72, 128)
    _zero_acc_slice(ones_v, acc, s)

    one = jnp.ones((16,), jnp.float32)

    def fill(i, _):
        for k in range(128 // 16):
            ones_v[i, pl.ds(k * 16, 16)] = one
        return 0

    lax.fori_loop(0, CHUNK, fill, 0)
    plsc.subcore_barrier()

    def chunk(j, _):
        pltpu.sync_copy(ones_v, acc.at[dst_v.at[j]], add=True)
        return 0

    lax.fori_loop(0, NCH, chunk, 0)
    plsc.subcore_barrier()

    pltpu.sync_copy(acc.at[pl.ds(s * RPT, RPT)],
                    out_hbm.at[c, pl.ds(s * RPT, RPT)])


@functools.lru_cache(maxsize=None)
def _make_deg_kernel():
    return pl.kernel(
        _deg_body,
        out_type=jax.ShapeDtypeStruct((NC, NPAD, 128), jnp.float32),
        mesh=_sc_mesh(),
        scratch_types=[
            pltpu.VMEM((NCH, CHUNK), jnp.int32),
            pltpu.VMEM((CHUNK, 128), jnp.float32),
            pltpu.VMEM_SHARED((NPAD, 128), jnp.float32),
        ],
    )


BLK = 1000


def _tc_first_body(x_ref, w_ref, degp_ref, y_ref, dinv_ref):
    deg = degp_ref[0, :, 0:1] + degp_ref[1, :, 0:1] + 1.0
    dinv = lax.rsqrt(deg)
    dinv_ref[...] = jnp.broadcast_to(dinv, dinv_ref.shape)
    y_ref[...] = jnp.dot(x_ref[...], w_ref[...],
                         preferred_element_type=jnp.float32) * dinv


def _tc_mid_body(part_ref, y_ref, b_ref, w_ref, dinv8_ref, out_ref):
    dinv = dinv8_ref[:, 0:1]
    agg = dinv * (part_ref[0] + part_ref[1] + y_ref[...]) + b_ref[...]
    h = jnp.maximum(agg, 0.0)
    out_ref[...] = jnp.dot(h, w_ref[...],
                           preferred_element_type=jnp.float32) * dinv


def _tc_pre_out_body(part_ref, y_ref, b_ref, w_ref, dinv8_ref, y2_ref, z_ref):
    dinv = dinv8_ref[:, 0:1]
    agg = dinv * (part_ref[0] + part_ref[1] + y_ref[...]) + b_ref[0:1, :]
    h = jnp.maximum(agg, 0.0)
    u = jnp.dot(h, w_ref[...], preferred_element_type=jnp.float32)
    y2_ref[...] = u[:, :128] * dinv
    z_ref[...] = u[:, 128:] + b_ref[1:2, :]


def _tc_out_body(part_ref, y_ref, z_ref, b_ref, dinv8_ref, out_ref):
    dinv = dinv8_ref[:, 0:1]
    agg = dinv * (part_ref[0] + part_ref[1] + y_ref[...]) + b_ref[...]
    out_ref[...] = (agg + z_ref[...])[:, :40]


def _mat_spec(D):
    return pl.BlockSpec((BLK, D), lambda i: (i, 0))


def _full_spec(shape):
    nd = len(shape)
    return pl.BlockSpec(shape, lambda i: (0,) * nd)


def _part_spec(D):
    return pl.BlockSpec((2, BLK, D), lambda i: (0, i, 0))


def _tc_first(x, w, degp):
    return pl.pallas_call(
        _tc_first_body,
        grid=(N // BLK,),
        in_specs=[_mat_spec(128), _full_spec((128, 128)),
                  pl.BlockSpec((2, BLK, 128), lambda i: (0, i, 0))],
        out_specs=[_mat_spec(128), _mat_spec(8)],
        out_shape=[jax.ShapeDtypeStruct((N, 128), jnp.float32),
                   jax.ShapeDtypeStruct((N, 8), jnp.float32)],
    )(x, w, degp)


def _tc_mid(part, y, b, w, dinv8):
    return pl.pallas_call(
        _tc_mid_body,
        grid=(N // BLK,),
        in_specs=[_part_spec(128), _mat_spec(128), _full_spec((1, 128)),
                  _full_spec((128, 128)), _mat_spec(8)],
        out_specs=_mat_spec(128),
        out_shape=jax.ShapeDtypeStruct((N, 128), jnp.float32),
    )(part, y, b, w, dinv8)


def _tc_pre_out(part, y, b, w, dinv8):
    return pl.pallas_call(
        _tc_pre_out_body,
        grid=(N // BLK,),
        in_specs=[_part_spec(128), _mat_spec(128), _full_spec((2, 128)),
                  _full_spec((128, 256)), _mat_spec(8)],
        out_specs=[_mat_spec(128), _mat_spec(128)],
        out_shape=[jax.ShapeDtypeStruct((N, 128), jnp.float32),
                   jax.ShapeDtypeStruct((N, 128), jnp.float32)],
    )(part, y, b, w, dinv8)


def _tc_out(part, y, z, b, dinv8):
    return pl.pallas_call(
        _tc_out_body,
        grid=(N // BLK,),
        in_specs=[_part_spec(128), _mat_spec(128), _mat_spec(128),
                  _full_spec((1, 128)), _mat_spec(8)],
        out_specs=pl.BlockSpec((BLK, 40), lambda i: (i, 0)),
        out_shape=jax.ShapeDtypeStruct((N, 40), jnp.float32),
    )(part, y, z, b, dinv8)


def kernel(x, edge_index, W0, b0, W1, b1, W2, b2, Ws, bs):
    src_w = jnp.pad(edge_index[0].reshape(NW, EW), ((0, 0), (0, EWP - EW)))
    dst_w = jnp.pad(edge_index[1].reshape(NW, EW), ((0, 0), (0, EWP - EW)),
                    constant_values=NPAD - 8)
    src = src_w.reshape(NW, NG, G, ECH)
    dst = dst_w.reshape(NW, NG, G, ECH)
    dst_flat = edge_index[1].reshape(NW, NCH, CHUNK)

    degp = _make_deg_kernel()(dst_flat)

    y0, dinv8 = _tc_first(x, W0, degp)
    p0 = _make_edge_scatter(128)(y0, src, dst)

    y1 = _tc_mid(p0, y0, b0.reshape(1, 128), W1, dinv8)
    p1 = _make_edge_scatter(128)(y1, src, dst)

    W2p = jnp.zeros((128, 128), jnp.float32).at[:, :40].set(W2)
    Wsp = jnp.zeros((128, 128), jnp.float32).at[:, :40].set(Ws)
    wcat = jnp.concatenate([W2p, Wsp], axis=1)
    bcat = jnp.zeros((2, 128), jnp.float32).at[0, :].set(b1).at[1, :40].set(bs)
    y2, z = _tc_pre_out(p1, y1, bcat, wcat, dinv8)
    p2 = _make_edge_scatter(128)(y2, src, dst)

    b2p = jnp.zeros((1, 128), jnp.float32).at[0, :40].set(b2)
    out = _tc_out(p2, y2, z, b2p, dinv8)
    return out


# speedup vs baseline: 1.5630x; 1.5630x over previous
import functools
import jax
import jax.numpy as jnp
from jax import lax
from jax.experimental import pallas as pl
from jax.experimental.pallas import tpu as pltpu
from jax.experimental.pallas import tpu_sc as plsc

N = 10000
NPAD = 10112
E = 320000
NC = 2
NS = 16
NW = NC * NS
EW = E // NW
CHUNK = 100
NCH = EW // CHUNK
ECH = 100
G = 10
NG = EW // ECH // G
RPT = NPAD // NS


@functools.lru_cache(maxsize=None)
def _sc_mesh():
    return plsc.VectorSubcoreMesh(core_axis_name="c", subcore_axis_name="s",
                                  num_cores=NC, num_subcores=NS)


def _memset_zero(ref, nrows, width):
    z = jnp.zeros((16,), jnp.float32)

    def body(i, _):
        for k in range(width // 16):
            ref[i, pl.ds(k * 16, 16)] = z
        return 0

    lax.fori_loop(0, nrows, body, 0)


def _zero_acc_slice(zsrc, acc, s):
    for j in range(8):
        pltpu.sync_copy(zsrc.at[pl.ds(0, 72)],
                        acc.at[pl.ds(s * RPT + j * 72, 72)])
    pltpu.sync_copy(zsrc.at[pl.ds(0, 56)],
                    acc.at[pl.ds(s * RPT + 576, 56)])


def _edge_scatter_body(D, y_hbm, src_hbm, dst_hbm, out_hbm,
                       srcg_a, srcg_b, dstg_a, dstg_b,
                       rows0, rows1, rows2, acc,
                       sem_ga, sem_gb,
                       sem_g0, sem_g1, sem_g2,
                       sem_s0, sem_s1, sem_s2):
    c = lax.axis_index("c")
    s = lax.axis_index("s")
    wid = s * NC + c
    rows = (rows0, rows1, rows2)
    sem_g = (sem_g0, sem_g1, sem_g2)
    sem_s = (sem_s0, sem_s1, sem_s2)

    def fire_idx(g, srcg, dstg, sem):
        pltpu.async_copy(src_hbm.at[wid, g], srcg, sem)
        pltpu.async_copy(dst_hbm.at[wid, g], dstg, sem)

    def wait_idx(g, srcg, dstg, sem):
        pltpu.make_async_copy(src_hbm.at[wid, g], srcg, sem).wait()
        pltpu.make_async_copy(dst_hbm.at[wid, g], dstg, sem).wait()

    fire_idx(0, srcg_a, dstg_a, sem_ga)

    _memset_zero(rows0, 72, D)
    _zero_acc_slice(rows0, acc, s)
    plsc.subcore_barrier()
    wait_idx(0, srcg_a, dstg_a, sem_ga)

    def process_group(srcg, dstg):
        def fire_g(j):
            b = j % 3
            pltpu.async_copy(y_hbm.at[srcg.at[j]], rows[b], sem_g[b])

        def wait_g(j):
            b = j % 3
            pltpu.make_async_copy(y_hbm.at[srcg.at[j]], rows[b],
                                  sem_g[b]).wait()

        def fire_s(j):
            b = j % 3
            pltpu.async_copy(rows[b], acc.at[dstg.at[j]], sem_s[b],
                             add=True)

        def wait_s(j):
            b = j % 3
            pltpu.make_async_copy(rows[b], acc.at[dstg.at[j]],
                                  sem_s[b]).wait()

        fire_g(0)
        fire_g(1)
        for j in range(G):
            wait_g(j)
            if j - 1 >= 0:
                wait_s(j - 1)
            fire_s(j)
            if j + 2 < G:
                fire_g(j + 2)
        wait_s(G - 1)

    def gpair(g2, _):
        g = g2 * 2
        fire_idx(g + 1, srcg_b, dstg_b, sem_gb)
        process_group(srcg_a, dstg_a)
        wait_idx(g + 1, srcg_b, dstg_b, sem_gb)

        @pl.when(g2 < NG // 2 - 1)
        def _():
            fire_idx(g + 2, srcg_a, dstg_a, sem_ga)

        process_group(srcg_b, dstg_b)

        @pl.when(g2 < NG // 2 - 1)
        def _():
            wait_idx(g + 2, srcg_a, dstg_a, sem_ga)

        return 0

    lax.fori_loop(0, NG // 2, gpair, 0)
    plsc.subcore_barrier()

    pltpu.sync_copy(acc.at[pl.ds(s * RPT, RPT)],
                    out_hbm.at[c, pl.ds(s * RPT, RPT)])


@functools.lru_cache(maxsize=None)
def _make_edge_scatter(D):
    body = functools.partial(_edge_scatter_body, D)
    return pl.kernel(
        body,
        out_type=jax.ShapeDtypeStruct((NC, NPAD, D), jnp.float32),
        mesh=_sc_mesh(),
        scratch_types=[
            pltpu.VMEM((G, ECH), jnp.int32),
            pltpu.VMEM((G, ECH), jnp.int32),
            pltpu.VMEM((G, ECH), jnp.int32),
            pltpu.VMEM((G, ECH), jnp.int32),
            pltpu.VMEM((ECH, D), jnp.float32),
            pltpu.VMEM((ECH, D), jnp.float32),
            pltpu.VMEM((ECH, D), jnp.float32),
            pltpu.VMEM_SHARED((NPAD, D), jnp.float32),
            pltpu.SemaphoreType.DMA,
            pltpu.SemaphoreType.DMA,
            pltpu.SemaphoreType.DMA,
            pltpu.SemaphoreType.DMA,
            pltpu.SemaphoreType.DMA,
            pltpu.SemaphoreType.DMA,
            pltpu.SemaphoreType.DMA,
            pltpu.SemaphoreType.DMA,
        ],
    )


def _deg_body(dst_hbm, out_hbm, dst_v, ones_v, acc):
    c = lax.axis_index("c")
    s = lax.axis_index("s")
    wid = s * NC + c

    pltpu.sync_copy(dst_hbm.at[wid], dst_v)

    _memset_zero(ones_v, 72, 128)
    _zero_acc_slice(ones_v, acc, s)

    one = jnp.ones((16,), jnp.float32)

    def fill(i, _):
        for k in range(128 // 16):
            ones_v[i, pl.ds(k * 16, 16)] = one
        return 0

    lax.fori_loop(0, CHUNK, fill, 0)
    plsc.subcore_barrier()

    def chunk(j, _):
        pltpu.sync_copy(ones_v, acc.at[dst_v.at[j]], add=True)
        return 0

    lax.fori_loop(0, NCH, chunk, 0)
    plsc.subcore_barrier()

    pltpu.sync_copy(acc.at[pl.ds(s * RPT, RPT)],
                    out_hbm.at[c, pl.ds(s * RPT, RPT)])


@functools.lru_cache(maxsize=None)
def _make_deg_kernel():
    return pl.kernel(
        _deg_body,
        out_type=jax.ShapeDtypeStruct((NC, NPAD, 128), jnp.float32),
        mesh=_sc_mesh(),
        scratch_types=[
            pltpu.VMEM((NCH, CHUNK), jnp.int32),
            pltpu.VMEM((CHUNK, 128), jnp.float32),
            pltpu.VMEM_SHARED((NPAD, 128), jnp.float32),
        ],
    )


BLK = 1000


def _tc_first_body(x_ref, w_ref, degp_ref, y_ref, dinv_ref):
    deg = degp_ref[0, :, 0:1] + degp_ref[1, :, 0:1] + 1.0
    dinv = lax.rsqrt(deg)
    dinv_ref[...] = jnp.broadcast_to(dinv, dinv_ref.shape)
    y_ref[...] = jnp.dot(x_ref[...], w_ref[...],
                         preferred_element_type=jnp.float32) * dinv


def _tc_mid_body(part_ref, y_ref, b_ref, w_ref, dinv8_ref, out_ref):
    dinv = dinv8_ref[:, 0:1]
    agg = dinv * (part_ref[0] + part_ref[1] + y_ref[...]) + b_ref[...]
    h = jnp.maximum(agg, 0.0)
    out_ref[...] = jnp.dot(h, w_ref[...],
                           preferred_element_type=jnp.float32) * dinv


def _tc_pre_out_body(part_ref, y_ref, b_ref, w_ref, dinv8_ref, y2_ref, z_ref):
    dinv = dinv8_ref[:, 0:1]
    agg = dinv * (part_ref[0] + part_ref[1] + y_ref[...]) + b_ref[0:1, :]
    h = jnp.maximum(agg, 0.0)
    u = jnp.dot(h, w_ref[...], preferred_element_type=jnp.float32)
    y2_ref[...] = u[:, :128] * dinv
    z_ref[...] = u[:, 128:] + b_ref[1:2, :]


def _tc_out_body(part_ref, y_ref, z_ref, b_ref, dinv8_ref, out_ref):
    dinv = dinv8_ref[:, 0:1]
    agg = dinv * (part_ref[0] + part_ref[1] + y_ref[...]) + b_ref[...]
    out_ref[...] = (agg + z_ref[...])[:, :40]


def _mat_spec(D):
    return pl.BlockSpec((BLK, D), lambda i: (i, 0))


def _full_spec(shape):
    nd = len(shape)
    return pl.BlockSpec(shape, lambda i: (0,) * nd)


def _part_spec(D):
    return pl.BlockSpec((2, BLK, D), lambda i: (0, i, 0))


def _tc_first(x, w, degp):
    return pl.pallas_call(
        _tc_first_body,
        grid=(N // BLK,),
        in_specs=[_mat_spec(128), _full_spec((128, 128)),
                  pl.BlockSpec((2, BLK, 128), lambda i: (0, i, 0))],
        out_specs=[_mat_spec(128), _mat_spec(8)],
        out_shape=[jax.ShapeDtypeStruct((N, 128), jnp.float32),
                   jax.ShapeDtypeStruct((N, 8), jnp.float32)],
    )(x, w, degp)


def _tc_mid(part, y, b, w, dinv8):
    return pl.pallas_call(
        _tc_mid_body,
        grid=(N // BLK,),
        in_specs=[_part_spec(128), _mat_spec(128), _full_spec((1, 128)),
                  _full_spec((128, 128)), _mat_spec(8)],
        out_specs=_mat_spec(128),
        out_shape=jax.ShapeDtypeStruct((N, 128), jnp.float32),
    )(part, y, b, w, dinv8)


def _tc_pre_out(part, y, b, w, dinv8):
    return pl.pallas_call(
        _tc_pre_out_body,
        grid=(N // BLK,),
        in_specs=[_part_spec(128), _mat_spec(128), _full_spec((2, 128)),
                  _full_spec((128, 256)), _mat_spec(8)],
        out_specs=[_mat_spec(128), _mat_spec(128)],
        out_shape=[jax.ShapeDtypeStruct((N, 128), jnp.float32),
                   jax.ShapeDtypeStruct((N, 128), jnp.float32)],
    )(part, y, b, w, dinv8)


def _tc_out(part, y, z, b, dinv8):
    return pl.pallas_call(
        _tc_out_body,
        grid=(N // BLK,),
        in_specs=[_part_spec(128), _mat_spec(128), _mat_spec(128),
                  _full_spec((1, 128)), _mat_spec(8)],
        out_specs=pl.BlockSpec((BLK, 40), lambda i: (i, 0)),
        out_shape=jax.ShapeDtypeStruct((N, 40), jnp.float32),
    )(part, y, z, b, dinv8)


def kernel(x, edge_index, W0, b0, W1, b1, W2, b2, Ws, bs):
    src = edge_index[0].reshape(NW, NG, G, ECH)
    dst = edge_index[1].reshape(NW, NG, G, ECH)
    dst_flat = edge_index[1].reshape(NW, NCH, CHUNK)

    degp = _make_deg_kernel()(dst_flat)

    y0, dinv8 = _tc_first(x, W0, degp)
    p0 = _make_edge_scatter(128)(y0, src, dst)

    y1 = _tc_mid(p0, y0, b0.reshape(1, 128), W1, dinv8)
    p1 = _make_edge_scatter(128)(y1, src, dst)

    W2p = jnp.zeros((128, 128), jnp.float32).at[:, :40].set(W2)
    Wsp = jnp.zeros((128, 128), jnp.float32).at[:, :40].set(Ws)
    wcat = jnp.concatenate([W2p, Wsp], axis=1)
    bcat = jnp.zeros((2, 128), jnp.float32).at[0, :].set(b1).at[1, :40].set(bs)
    y2, z = _tc_pre_out(p1, y1, bcat, wcat, dinv8)
    p2 = _make_edge_scatter(128)(y2, src, dst)

    b2p = jnp.zeros((1, 128), jnp.float32).at[0, :40].set(b2)
    out = _tc_out(p2, y2, z, b2p, dinv8)
    return out
